# baseline (speedup 1.0000x reference)
import functools

import jax
import jax.numpy as jnp
from jax import lax
from jax.experimental import pallas as pl
from jax.experimental.pallas import tpu as pltpu
from jax.experimental.pallas import tpu_sc as plsc

_NUM_EMB = 1000000
_DIM = 64
_BATCH = 16384
_NC = 2
_NS = 16
_NW = _NC * _NS
_BPW = _BATCH // _NW
_CHUNK = 128
_NCHUNK = _BPW // _CHUNK

_mesh = plsc.VectorSubcoreMesh(core_axis_name="c", subcore_axis_name="s")


@functools.partial(
    pl.kernel,
    mesh=_mesh,
    out_type=jax.ShapeDtypeStruct((_DIM, _BATCH), jnp.float32),
    scratch_types=[
        pltpu.VMEM((_BPW,), jnp.int32),
        pltpu.VMEM((_BPW,), jnp.int32),
        pltpu.VMEM((_NCHUNK, _CHUNK, 128), jnp.float32),
        pltpu.VMEM((_DIM, _BPW), jnp.float32),
        pltpu.SemaphoreType.DMA,
    ],
    compiler_params=pltpu.CompilerParams(
        use_tc_tiling_on_sc=True, needs_layout_passes=False
    ),
)
def _gather_kernel(idx_hbm, tab_hbm, out_hbm, idx_v, half_v, rows_v, out_v, sem):
    wid = lax.axis_index("s") * _NC + lax.axis_index("c")
    base = wid * _BPW
    pltpu.sync_copy(idx_hbm.at[pl.ds(base, _BPW)], idx_v)

    for v in range(_BPW // 16):
        idx_v16 = idx_v[pl.ds(v * 16, 16)]
        half_v[pl.ds(v * 16, 16)] = idx_v16 >> 1

    copies = [
        pltpu.async_copy(
            tab_hbm.at[half_v.at[pl.ds(c * _CHUNK, _CHUNK)]],
            rows_v.at[c],
            sem,
        )
        for c in range(_NCHUNK)
    ]
    for cp in copies:
        cp.wait()

    iota16 = lax.iota(jnp.int32, 16)

    def group_body(g, carry):
        row16 = iota16 + (g & (_CHUNK // 16 - 1)) * 16
        chunk = g >> 3
        parity = (idx_v[pl.ds(g * 16, 16)] & 1) << 6
        for f in range(_DIM):
            vals = plsc.load_gather(
                rows_v, [jnp.full((16,), chunk, jnp.int32), row16, parity + f]
            )
            out_v[f, pl.ds(g * 16, 16)] = vals
        return carry

    lax.fori_loop(0, _BPW // 16, group_body, 0)
    pltpu.sync_copy(out_v, out_hbm.at[:, pl.ds(base, _BPW)])


def kernel(indices, drug_feature):
    idx = indices.astype(jnp.int32)
    tab = drug_feature.reshape(_NUM_EMB // 2, 2 * _DIM)
    out_t = _gather_kernel(idx, tab)
    return out_t.T
